# baseline (speedup 1.0000x reference)
import functools

import jax
import jax.numpy as jnp
from jax import lax
from jax.experimental import pallas as pl
from jax.experimental.pallas import tpu as pltpu
from jax.experimental.pallas import tpu_sc as plsc

_H = 224
_W = 224
_HP = 256
_B = 4
_NS = 16
_ROWS = _HP // _NS
_CHUNKS = _ROWS * (_W // 16)
_CAP = _ROWS * _W + 16
_GCAP = _H * _W + 256
_SENT = 1.0e6
_BIGD = 1.0e13


def _shift(x, di, dj, fill, ii, jj):
    y = x
    if di:
        y = jnp.roll(y, -di, axis=0)
    if dj:
        y = jnp.roll(y, -dj, axis=1)
    ok = jnp.full(x.shape, True)
    if di:
        ok = ok & (ii + di >= 0) & (ii + di < _H)
    if dj:
        ok = ok & (jj + dj >= 0) & (jj + dj < _W)
    return jnp.where(ok, y, fill)


def _corner_mask(img, ii, jj):
    sh = lambda x, di, dj: _shift(x, di, dj, 0.0, ii, jj)
    ix = (-sh(img, -1, -1) + sh(img, -1, 1)
          - 2.0 * sh(img, 0, -1) + 2.0 * sh(img, 0, 1)
          - sh(img, 1, -1) + sh(img, 1, 1))
    iy = (-sh(img, -1, -1) - 2.0 * sh(img, -1, 0) - sh(img, -1, 1)
          + sh(img, 1, -1) + 2.0 * sh(img, 1, 0) + sh(img, 1, 1))
    ixx = ix * ix
    iyy = iy * iy
    ixy = ix * iy
    sxx = jnp.zeros_like(img)
    syy = jnp.zeros_like(img)
    sxy = jnp.zeros_like(img)
    for di in (-1, 0, 1):
        for dj in (-1, 0, 1):
            sxx = sxx + sh(ixx, di, dj)
            syy = syy + sh(iyy, di, dj)
            sxy = sxy + sh(ixy, di, dj)
    tr = sxx + syy
    disc = (sxx - syy) * (sxx - syy) + 4.0 * sxy * sxy
    eig = 0.5 * (tr - jnp.sqrt(jnp.maximum(disc, 0.0)))
    lmax = eig
    for di in range(-2, 3):
        for dj in range(-2, 3):
            if di == 0 and dj == 0:
                continue
            lmax = jnp.maximum(lmax, _shift(eig, di, dj, -jnp.inf, ii, jj))
    thresh = 0.01 * jnp.max(eig)
    mask = (eig >= lmax) & (eig > thresh)
    pe = jnp.where(mask, eig, 0.0)
    return (pe != 0.0).astype(jnp.float32)


def _masks_body(pred_ref, tgt_ref, out_ref):
    ii = lax.broadcasted_iota(jnp.int32, (_H, _W), 0)
    jj = lax.broadcasted_iota(jnp.int32, (_H, _W), 1)
    img_p = (pred_ref[0, 0] > 0.0).astype(jnp.float32)
    out_ref[0, 0, pl.ds(0, _H), :] = _corner_mask(img_p, ii, jj)
    out_ref[0, 1, pl.ds(0, _H), :] = _corner_mask(tgt_ref[0, 0], ii, jj)
    pad = jnp.zeros((_HP - _H, _W), jnp.float32)
    out_ref[0, 0, pl.ds(_H, _HP - _H), :] = pad
    out_ref[0, 1, pl.ds(_H, _HP - _H), :] = pad


def _newton_sqrt(a):
    i = lax.bitcast_convert_type(a, jnp.int32)
    i = jnp.int32(0x1FBD1DF5) + lax.shift_right_logical(i, 1)
    y = lax.bitcast_convert_type(i, jnp.float32)
    for _ in range(3):
        y = 0.5 * (y + a / y)
    return y


def _sc_body(masks, out, strip, lx, ly, tx, ty, cx, cy, cnts_l, vbuf,
             sh_cnts, sh_x, sh_y):
    cid = lax.axis_index("c")
    sid = lax.axis_index("s")
    lane = lax.iota(jnp.int32, 16)
    lane_f = lane.astype(jnp.float32)

    def compact(which, b, dst_x, dst_y):
        pltpu.sync_copy(masks.at[b, which, pl.ds(sid * _ROWS, _ROWS)], strip)

        def chunk(k, cnt):
            r = k // 14
            cb = (k % 14) * 16
            vals = strip[r, pl.ds(cb, 16)]
            m = vals != 0.0
            xs = (r + sid * _ROWS).astype(jnp.float32) + 0.0 * lane_f
            ys = cb.astype(jnp.float32) + lane_f
            mi = m.astype(jnp.int32)
            incl = plsc.cumsum(mi)
            idx = cnt + incl - mi
            plsc.store_scatter(dst_x, [idx], xs, mask=m)
            plsc.store_scatter(dst_y, [idx], ys, mask=m)
            return cnt + incl[15]

        cnt = lax.fori_loop(0, _CHUNKS, chunk, jnp.int32(0))
        sent = jnp.full((16,), _SENT, jnp.float32)
        all_on = jnp.full((16,), True)
        plsc.store_scatter(dst_x, [cnt + lane], sent, mask=all_on)
        plsc.store_scatter(dst_y, [cnt + lane], sent, mask=all_on)
        return cnt

    def upload(src_x, src_y, my_pad, c_off):
        def up(t, _):
            off = pl.multiple_of(c_off + t * 16, 16)
            src = pl.multiple_of(t * 16, 16)
            pltpu.sync_copy(src_x.at[pl.ds(src, 16)],
                            sh_x.at[pl.ds(off, 16)])
            pltpu.sync_copy(src_y.at[pl.ds(src, 16)],
                            sh_y.at[pl.ds(off, 16)])
            return 0
        lax.fori_loop(0, my_pad // 16, up, 0)

    def download(pad_n):
        def dn(t, _):
            off = pl.multiple_of(t * 256, 256)
            pltpu.sync_copy(sh_x.at[pl.ds(off, 256)],
                            cx.at[pl.ds(off, 256)])
            pltpu.sync_copy(sh_y.at[pl.ds(off, 256)],
                            cy.at[pl.ds(off, 256)])
            return 0
        lax.fori_loop(0, (pad_n + 255) // 256, dn, 0)

    def chamfer_dir(qx_ref, qy_ref, q_cnt, n_cand):
        q_pad = ((q_cnt + 15) // 16) * 16

        def per_chunk(qc, acc):
            qx = qx_ref[pl.ds(qc * 16, 16)]
            qy = qy_ref[pl.ds(qc * 16, 16)]

            def per_cand_chunk(jc, best):
                cvx = cx[pl.ds(jc * 16, 16)]
                cvy = cy[pl.ds(jc * 16, 16)]
                for l in range(16):
                    dx = qx - jnp.broadcast_to(cvx[l], (16,))
                    dy = qy - jnp.broadcast_to(cvy[l], (16,))
                    best = jnp.minimum(best, dx * dx + dy * dy)
                return best

            best = lax.fori_loop(0, n_cand // 16, per_cand_chunk,
                                 jnp.full((16,), _BIGD, jnp.float32))
            s = _newton_sqrt(jnp.maximum(best, 1e-6))
            contrib = jnp.where(qx < _SENT * 0.5, s, 0.0)
            return acc + jnp.sum(contrib)

        return lax.fori_loop(0, q_pad // 16, per_chunk, jnp.float32(0.0))

    def process_batch(bb, _):
        b = cid * 2 + bb
        cnt_p = compact(0, b, lx, ly)
        cnt_t = compact(1, b, tx, ty)

        cvec = jnp.where(lane == 0, cnt_p,
                         jnp.where(lane == 1, cnt_t, 0)).astype(jnp.int32)
        vbuf[...] = lax.bitcast_convert_type(cvec, jnp.float32)
        pltpu.sync_copy(vbuf, sh_cnts.at[sid])
        plsc.subcore_barrier()
        pltpu.sync_copy(sh_cnts, cnts_l)

        def scan_counts(i, carry):
            off_p, off_t, pad_p, pad_t = carry
            row = lax.bitcast_convert_type(cnts_l[i], jnp.int32)
            cp = jnp.sum(jnp.where(lane == 0, row, 0))
            ct = jnp.sum(jnp.where(lane == 1, row, 0))
            pp = ((cp + 15) // 16) * 16
            pt = ((ct + 15) // 16) * 16
            before = (i < sid).astype(jnp.int32)
            return (off_p + before * pp, off_t + before * pt,
                    pad_p + pp, pad_t + pt)

        z = jnp.int32(0)
        off_p, off_t, pad_p, pad_t = lax.fori_loop(
            0, _NS, scan_counts, (z, z, z, z))
        my_pad_p = ((cnt_p + 15) // 16) * 16
        my_pad_t = ((cnt_t + 15) // 16) * 16

        upload(lx, ly, my_pad_p, off_p)
        plsc.subcore_barrier()
        download(pad_p)
        accA = chamfer_dir(tx, ty, cnt_t, pad_p)
        plsc.subcore_barrier()

        upload(tx, ty, my_pad_t, off_t)
        plsc.subcore_barrier()
        download(pad_t)
        accB = chamfer_dir(lx, ly, cnt_p, pad_t)
        plsc.subcore_barrier()

        ovec = jnp.where(
            lane == 0, accA,
            jnp.where(lane == 1, accB,
                      jnp.where(lane == 2, cnt_p.astype(jnp.float32),
                                jnp.where(lane == 3,
                                          cnt_t.astype(jnp.float32), 0.0))))
        vbuf[...] = ovec
        pltpu.sync_copy(vbuf, out.at[b, sid])
        return 0

    lax.fori_loop(0, 2, process_batch, 0)


def _finish_body(parts_ref, out_ref):
    x = parts_ref[...]
    sumA = jnp.sum(x[:, :, 0], axis=1)
    sumB = jnp.sum(x[:, :, 1], axis=1)
    cp = jnp.sum(x[:, :, 2], axis=1)
    ct = jnp.sum(x[:, :, 3], axis=1)
    m1 = sumA / jnp.maximum(ct, 1.0)
    m2 = sumB / jnp.maximum(cp, 1.0)
    valid = (cp > 0.0) & (ct > 0.0)
    loss = jnp.sum(jnp.where(valid, 0.5 * (m1 + m2), 0.0))
    out_ref[...] = jnp.full((1, 1), loss)


def kernel(pred, target):
    if pred.ndim == 3:
        pred = pred[:, None]
    if target.ndim == 3:
        target = target[:, None]
    masks = pl.pallas_call(
        _masks_body,
        grid=(_B,),
        in_specs=[
            pl.BlockSpec((1, 1, _H, _W), lambda b: (b, 0, 0, 0)),
            pl.BlockSpec((1, 1, _H, _W), lambda b: (b, 0, 0, 0)),
        ],
        out_specs=pl.BlockSpec((1, 2, _HP, _W), lambda b: (b, 0, 0, 0)),
        out_shape=jax.ShapeDtypeStruct((_B, 2, _HP, _W), jnp.float32),
    )(pred.astype(jnp.float32), target.astype(jnp.float32))

    sc = pl.kernel(
        _sc_body,
        mesh=plsc.VectorSubcoreMesh(core_axis_name="c", subcore_axis_name="s"),
        compiler_params=pltpu.CompilerParams(needs_layout_passes=False),
        out_type=jax.ShapeDtypeStruct((_B, _NS, 16), jnp.float32),
        scratch_types=[
            pltpu.VMEM((_ROWS, _W), jnp.float32),
            pltpu.VMEM((_CAP,), jnp.float32),
            pltpu.VMEM((_CAP,), jnp.float32),
            pltpu.VMEM((_CAP,), jnp.float32),
            pltpu.VMEM((_CAP,), jnp.float32),
            pltpu.VMEM((_GCAP,), jnp.float32),
            pltpu.VMEM((_GCAP,), jnp.float32),
            pltpu.VMEM((_NS, 16), jnp.float32),
            pltpu.VMEM((16,), jnp.float32),
            pltpu.VMEM_SHARED((_NS, 16), jnp.float32),
            pltpu.VMEM_SHARED((_GCAP,), jnp.float32),
            pltpu.VMEM_SHARED((_GCAP,), jnp.float32),
        ],
    )(masks)

    out = pl.pallas_call(
        _finish_body,
        out_shape=jax.ShapeDtypeStruct((1, 1), jnp.float32),
    )(sc)
    return out.reshape(1)


# speedup vs baseline: 1.3036x; 1.3036x over previous
import functools

import jax
import jax.numpy as jnp
from jax import lax
from jax.experimental import pallas as pl
from jax.experimental.pallas import tpu as pltpu
from jax.experimental.pallas import tpu_sc as plsc

_H = 224
_W = 224
_HP = 256
_B = 4
_NS = 16
_ROWS = _HP // _NS
_CHUNKS = _ROWS * (_W // 16)
_CAP = _ROWS * _W + 16
_GCAP = 50688
_SENT = 1.0e6
_BIGD = 1.0e13


def _shift(x, di, dj, fill, ii, jj):
    y = x
    if di:
        y = jnp.roll(y, -di, axis=0)
    if dj:
        y = jnp.roll(y, -dj, axis=1)
    ok = jnp.full(x.shape, True)
    if di:
        ok = ok & (ii + di >= 0) & (ii + di < _H)
    if dj:
        ok = ok & (jj + dj >= 0) & (jj + dj < _W)
    return jnp.where(ok, y, fill)


def _corner_mask(img, ii, jj):
    sh = lambda x, di, dj: _shift(x, di, dj, 0.0, ii, jj)
    ix = (-sh(img, -1, -1) + sh(img, -1, 1)
          - 2.0 * sh(img, 0, -1) + 2.0 * sh(img, 0, 1)
          - sh(img, 1, -1) + sh(img, 1, 1))
    iy = (-sh(img, -1, -1) - 2.0 * sh(img, -1, 0) - sh(img, -1, 1)
          + sh(img, 1, -1) + 2.0 * sh(img, 1, 0) + sh(img, 1, 1))
    ixx = ix * ix
    iyy = iy * iy
    ixy = ix * iy
    sxx = jnp.zeros_like(img)
    syy = jnp.zeros_like(img)
    sxy = jnp.zeros_like(img)
    for di in (-1, 0, 1):
        for dj in (-1, 0, 1):
            sxx = sxx + sh(ixx, di, dj)
            syy = syy + sh(iyy, di, dj)
            sxy = sxy + sh(ixy, di, dj)
    tr = sxx + syy
    disc = (sxx - syy) * (sxx - syy) + 4.0 * sxy * sxy
    eig = 0.5 * (tr - jnp.sqrt(jnp.maximum(disc, 0.0)))
    lmax = eig
    for di in range(-2, 3):
        for dj in range(-2, 3):
            if di == 0 and dj == 0:
                continue
            lmax = jnp.maximum(lmax, _shift(eig, di, dj, -jnp.inf, ii, jj))
    thresh = 0.01 * jnp.max(eig)
    mask = (eig >= lmax) & (eig > thresh)
    pe = jnp.where(mask, eig, 0.0)
    return (pe != 0.0).astype(jnp.float32)


def _masks_body(pred_ref, tgt_ref, out_ref):
    ii = lax.broadcasted_iota(jnp.int32, (_H, _W), 0)
    jj = lax.broadcasted_iota(jnp.int32, (_H, _W), 1)
    img_p = (pred_ref[0, 0] > 0.0).astype(jnp.float32)
    out_ref[0, 0, pl.ds(0, _H), :] = _corner_mask(img_p, ii, jj)
    out_ref[0, 1, pl.ds(0, _H), :] = _corner_mask(tgt_ref[0, 0], ii, jj)
    pad = jnp.zeros((_HP - _H, _W), jnp.float32)
    out_ref[0, 0, pl.ds(_H, _HP - _H), :] = pad
    out_ref[0, 1, pl.ds(_H, _HP - _H), :] = pad


def _newton_sqrt(a):
    i = lax.bitcast_convert_type(a, jnp.int32)
    i = jnp.int32(0x1FBD1DF5) + lax.shift_right_logical(i, 1)
    y = lax.bitcast_convert_type(i, jnp.float32)
    for _ in range(3):
        y = 0.5 * (y + a / y)
    return y


def _sc_body(masks, out, strip, lx, ly, tx, ty, cx, cy, cnts_l, vbuf,
             sh_cnts, sh_x, sh_y):
    cid = lax.axis_index("c")
    sid = lax.axis_index("s")
    lane = lax.iota(jnp.int32, 16)
    lane_f = lane.astype(jnp.float32)

    def compact(which, b, dst_x, dst_y):
        pltpu.sync_copy(masks.at[b, which, pl.ds(sid * _ROWS, _ROWS)], strip)

        def chunk(k, cnt):
            r = k // 14
            cb = (k % 14) * 16
            vals = strip[r, pl.ds(cb, 16)]
            m = vals != 0.0
            xs = (r + sid * _ROWS).astype(jnp.float32) + 0.0 * lane_f
            ys = cb.astype(jnp.float32) + lane_f
            mi = m.astype(jnp.int32)
            incl = plsc.cumsum(mi)
            idx = cnt + incl - mi
            plsc.store_scatter(dst_x, [idx], xs, mask=m)
            plsc.store_scatter(dst_y, [idx], ys, mask=m)
            return cnt + incl[15]

        cnt = lax.fori_loop(0, _CHUNKS, chunk, jnp.int32(0))
        sent = jnp.full((16,), _SENT, jnp.float32)
        all_on = jnp.full((16,), True)
        plsc.store_scatter(dst_x, [cnt + lane], sent, mask=all_on)
        plsc.store_scatter(dst_y, [cnt + lane], sent, mask=all_on)
        return cnt

    def upload(src_x, src_y, my_pad, c_off, dst_sx, dst_sy):
        def up(t, _):
            off = pl.multiple_of(c_off + t * 16, 16)
            src = pl.multiple_of(t * 16, 16)
            pltpu.sync_copy(src_x.at[pl.ds(src, 16)],
                            dst_sx.at[pl.ds(off, 16)])
            pltpu.sync_copy(src_y.at[pl.ds(src, 16)],
                            dst_sy.at[pl.ds(off, 16)])
            return 0
        lax.fori_loop(0, my_pad // 16, up, 0)

    def download(pad_n, src_sx, src_sy):
        def dn(t, _):
            off = pl.multiple_of(t * 512, 512)
            pltpu.sync_copy(src_sx.at[pl.ds(off, 512)],
                            cx.at[pl.ds(off, 512)])
            pltpu.sync_copy(src_sy.at[pl.ds(off, 512)],
                            cy.at[pl.ds(off, 512)])
            return 0
        lax.fori_loop(0, (pad_n + 511) // 512, dn, 0)

    def chamfer_dir(qx_ref, qy_ref, q_cnt, n_cand):
        nqc = (q_cnt + 15) // 16
        ncc = n_cand // 16

        def eval_chunk(jc, carry):
            b0, b1, b2, b3, ub, qs = carry
            (q0x, q0y), (q1x, q1y), (q2x, q2y), (q3x, q3y) = qs
            joff = pl.multiple_of(jc * 16, 16)
            cvx = cx[pl.ds(joff, 16)]
            cvy = cy[pl.ds(joff, 16)]
            for l in range(16):
                bx = jnp.broadcast_to(cvx[l], (16,))
                by = jnp.broadcast_to(cvy[l], (16,))
                d0x = q0x - bx
                d0y = q0y - by
                b0 = jnp.minimum(b0, d0x * d0x + d0y * d0y)
                d1x = q1x - bx
                d1y = q1y - by
                b1 = jnp.minimum(b1, d1x * d1x + d1y * d1y)
                d2x = q2x - bx
                d2y = q2y - by
                b2 = jnp.minimum(b2, d2x * d2x + d2y * d2y)
                d3x = q3x - bx
                d3y = q3y - by
                b3 = jnp.minimum(b3, d3x * d3x + d3y * d3y)
            ub = jnp.max(jnp.maximum(jnp.maximum(b0, b1),
                                     jnp.maximum(b2, b3)))
            return (b0, b1, b2, b3, ub, qs)

        def per_group(g, acc):
            qs = []
            for u in range(4):
                qi = jnp.minimum(g * 4 + u, nqc - 1)
                qoff = pl.multiple_of(qi * 16, 16)
                qs.append((qx_ref[pl.ds(qoff, 16)], qy_ref[pl.ds(qoff, 16)]))
            qs = tuple(qs)
            qmin = qs[0][0][0]
            mx = jnp.full((16,), 0.0)
            for u in range(4):
                mx = jnp.maximum(mx, jnp.where(qs[u][0] < _SENT * 0.5,
                                               qs[u][0], 0.0))
            qmax = jnp.max(mx)
            qmid = 0.5 * (qmin + qmax)

            def bs_cond(c):
                blo, bhi = c
                return bhi - blo > 1

            def bs_body(c):
                blo, bhi = c
                mid = (blo + bhi) // 2
                moff = pl.multiple_of(mid * 16, 16)
                xm = cx[pl.ds(moff, 16)][0]
                go = xm <= qmid
                return (jnp.where(go, mid, blo), jnp.where(go, bhi, mid))

            blo, _ = lax.while_loop(bs_cond, bs_body,
                                    (jnp.int32(0), ncc - 1))

            big = jnp.full((16,), _BIGD, jnp.float32)
            init = (big, big, big, big, jnp.float32(_BIGD), qs)

            def seed(carry):
                for dd in (-1, 0, 1):
                    jc = jnp.clip(blo + dd, 0, ncc - 1)
                    carry = eval_chunk(jc, carry)
                return carry

            carry = lax.cond(ncc > 0, seed, lambda c: c, init)

            def scan_chunk(jc, carry):
                joff = pl.multiple_of(jc * 16, 16)
                cvx = cx[pl.ds(joff, 16)]
                xlo = cvx[0]
                xhi = cvx[15]
                lb = jnp.maximum(jnp.maximum(xlo - qmax, qmin - xhi), 0.0)
                return lax.cond(lb * lb >= carry[4], lambda c: c,
                                lambda c: eval_chunk(jc, c), carry)

            b0, b1, b2, b3, _, _ = lax.fori_loop(0, ncc, scan_chunk, carry)

            dacc = jnp.float32(0.0)
            for u, b in enumerate((b0, b1, b2, b3)):
                s = _newton_sqrt(jnp.maximum(b, 1e-6))
                ok = (g * 4 + u < nqc) & (qs[u][0] < _SENT * 0.5)
                dacc = dacc + jnp.sum(jnp.where(ok, s, 0.0))
            return acc + dacc

        return lax.fori_loop(0, (nqc + 3) // 4, per_group, jnp.float32(0.0))

    def process_batch(bb, _):
        b = cid * 2 + bb
        cnt_p = compact(0, b, lx, ly)
        cnt_t = compact(1, b, tx, ty)

        cvec = jnp.where(lane == 0, cnt_p,
                         jnp.where(lane == 1, cnt_t, 0)).astype(jnp.int32)
        vbuf[...] = lax.bitcast_convert_type(cvec, jnp.float32)
        pltpu.sync_copy(vbuf, sh_cnts.at[sid])
        plsc.subcore_barrier()
        pltpu.sync_copy(sh_cnts, cnts_l)

        def scan_counts(i, carry):
            off_p, off_t, pad_p, pad_t = carry
            row = lax.bitcast_convert_type(cnts_l[i], jnp.int32)
            cp = jnp.sum(jnp.where(lane == 0, row, 0))
            ct = jnp.sum(jnp.where(lane == 1, row, 0))
            pp = ((cp + 15) // 16) * 16
            pt = ((ct + 15) // 16) * 16
            before = (i < sid).astype(jnp.int32)
            return (off_p + before * pp, off_t + before * pt,
                    pad_p + pp, pad_t + pt)

        z = jnp.int32(0)
        off_p, off_t, pad_p, pad_t = lax.fori_loop(
            0, _NS, scan_counts, (z, z, z, z))
        my_pad_p = ((cnt_p + 15) // 16) * 16
        my_pad_t = ((cnt_t + 15) // 16) * 16

        upload(lx, ly, my_pad_p, off_p, sh_x, sh_y)
        plsc.subcore_barrier()
        download(pad_p, sh_x, sh_y)
        accA = chamfer_dir(tx, ty, cnt_t, pad_p)
        plsc.subcore_barrier()

        upload(tx, ty, my_pad_t, off_t, sh_x, sh_y)
        plsc.subcore_barrier()
        download(pad_t, sh_x, sh_y)
        accB = chamfer_dir(lx, ly, cnt_p, pad_t)
        plsc.subcore_barrier()

        ovec = jnp.where(
            lane == 0, accA,
            jnp.where(lane == 1, accB,
                      jnp.where(lane == 2, cnt_p.astype(jnp.float32),
                                jnp.where(lane == 3,
                                          cnt_t.astype(jnp.float32), 0.0))))
        vbuf[...] = ovec
        pltpu.sync_copy(vbuf, out.at[b, sid])
        return 0

    lax.fori_loop(0, 2, process_batch, 0)


def _finish_body(parts_ref, out_ref):
    x = parts_ref[...]
    sumA = jnp.sum(x[:, :, 0], axis=1)
    sumB = jnp.sum(x[:, :, 1], axis=1)
    cp = jnp.sum(x[:, :, 2], axis=1)
    ct = jnp.sum(x[:, :, 3], axis=1)
    m1 = sumA / jnp.maximum(ct, 1.0)
    m2 = sumB / jnp.maximum(cp, 1.0)
    valid = (cp > 0.0) & (ct > 0.0)
    loss = jnp.sum(jnp.where(valid, 0.5 * (m1 + m2), 0.0))
    out_ref[...] = jnp.full((1, 1), loss)


def kernel(pred, target):
    if pred.ndim == 3:
        pred = pred[:, None]
    if target.ndim == 3:
        target = target[:, None]
    masks = pl.pallas_call(
        _masks_body,
        grid=(_B,),
        in_specs=[
            pl.BlockSpec((1, 1, _H, _W), lambda b: (b, 0, 0, 0)),
            pl.BlockSpec((1, 1, _H, _W), lambda b: (b, 0, 0, 0)),
        ],
        out_specs=pl.BlockSpec((1, 2, _HP, _W), lambda b: (b, 0, 0, 0)),
        out_shape=jax.ShapeDtypeStruct((_B, 2, _HP, _W), jnp.float32),
    )(pred.astype(jnp.float32), target.astype(jnp.float32))

    sc = pl.kernel(
        _sc_body,
        mesh=plsc.VectorSubcoreMesh(core_axis_name="c", subcore_axis_name="s"),
        compiler_params=pltpu.CompilerParams(needs_layout_passes=False),
        out_type=jax.ShapeDtypeStruct((_B, _NS, 16), jnp.float32),
        scratch_types=[
            pltpu.VMEM((_ROWS, _W), jnp.float32),
            pltpu.VMEM((_CAP,), jnp.float32),
            pltpu.VMEM((_CAP,), jnp.float32),
            pltpu.VMEM((_CAP,), jnp.float32),
            pltpu.VMEM((_CAP,), jnp.float32),
            pltpu.VMEM((_GCAP,), jnp.float32),
            pltpu.VMEM((_GCAP,), jnp.float32),
            pltpu.VMEM((_NS, 16), jnp.float32),
            pltpu.VMEM((16,), jnp.float32),
            pltpu.VMEM_SHARED((_NS, 16), jnp.float32),
            pltpu.VMEM_SHARED((_GCAP,), jnp.float32),
            pltpu.VMEM_SHARED((_GCAP,), jnp.float32),
        ],
    )(masks)

    out = pl.pallas_call(
        _finish_body,
        out_shape=jax.ShapeDtypeStruct((1, 1), jnp.float32),
    )(sc)
    return out.reshape(1)


# speedup vs baseline: 1.6696x; 1.2808x over previous
import functools

import jax
import jax.numpy as jnp
from jax import lax
from jax.experimental import pallas as pl
from jax.experimental.pallas import tpu as pltpu
from jax.experimental.pallas import tpu_sc as plsc

_H = 224
_W = 224
_HP = 256
_B = 4
_NS = 16
_ROWS = _HP // _NS
_CHUNKS = _ROWS * (_W // 16)
_CAP = _ROWS * _W + 16
_GCAP = 50688
_SENT = 1.0e6
_BIGD = 1.0e13


def _shift(x, di, dj, fill, ii, jj):
    y = x
    if di:
        y = jnp.roll(y, -di, axis=0)
    if dj:
        y = jnp.roll(y, -dj, axis=1)
    ok = jnp.full(x.shape, True)
    if di:
        ok = ok & (ii + di >= 0) & (ii + di < _H)
    if dj:
        ok = ok & (jj + dj >= 0) & (jj + dj < _W)
    return jnp.where(ok, y, fill)


def _corner_mask(img, ii, jj):
    sh = lambda x, di, dj: _shift(x, di, dj, 0.0, ii, jj)

    def box3(x):
        h = sh(x, 0, -1) + x + sh(x, 0, 1)
        return sh(h, -1, 0) + h + sh(h, 1, 0)

    hx = sh(img, 0, 1) - sh(img, 0, -1)
    ix = sh(hx, -1, 0) + 2.0 * hx + sh(hx, 1, 0)
    hy = sh(img, 1, 0) - sh(img, -1, 0)
    iy = sh(hy, 0, -1) + 2.0 * hy + sh(hy, 0, 1)
    sxx = box3(ix * ix)
    syy = box3(iy * iy)
    sxy = box3(ix * iy)
    tr = sxx + syy
    disc = (sxx - syy) * (sxx - syy) + 4.0 * sxy * sxy
    eig = 0.5 * (tr - jnp.sqrt(jnp.maximum(disc, 0.0)))
    hm = eig
    for dj in (-2, -1, 1, 2):
        hm = jnp.maximum(hm, _shift(eig, 0, dj, -jnp.inf, ii, jj))
    lmax = hm
    for di in (-2, -1, 1, 2):
        lmax = jnp.maximum(lmax, _shift(hm, di, 0, -jnp.inf, ii, jj))
    thresh = 0.01 * jnp.max(eig)
    mask = (eig >= lmax) & (eig > thresh)
    pe = jnp.where(mask, eig, 0.0)
    return (pe != 0.0).astype(jnp.float32)


def _masks_body(pred_ref, tgt_ref, out_ref):
    ii = lax.broadcasted_iota(jnp.int32, (_H, _W), 0)
    jj = lax.broadcasted_iota(jnp.int32, (_H, _W), 1)
    img_p = (pred_ref[0, 0] > 0.0).astype(jnp.float32)
    out_ref[0, 0, pl.ds(0, _H), :] = _corner_mask(img_p, ii, jj)
    out_ref[0, 1, pl.ds(0, _H), :] = _corner_mask(tgt_ref[0, 0], ii, jj)
    pad = jnp.zeros((_HP - _H, _W), jnp.float32)
    out_ref[0, 0, pl.ds(_H, _HP - _H), :] = pad
    out_ref[0, 1, pl.ds(_H, _HP - _H), :] = pad


def _newton_sqrt(a):
    i = lax.bitcast_convert_type(a, jnp.int32)
    i = jnp.int32(0x1FBD1DF5) + lax.shift_right_logical(i, 1)
    y = lax.bitcast_convert_type(i, jnp.float32)
    for _ in range(3):
        y = 0.5 * (y + a / y)
    return y


def _sc_body(masks, out, strip, lx, ly, tx, ty, cx, cy, cnts_l, vbuf,
             sh_cnts, sh_x, sh_y):
    cid = lax.axis_index("c")
    sid = lax.axis_index("s")
    lane = lax.iota(jnp.int32, 16)
    lane_f = lane.astype(jnp.float32)

    def compact(which, b, dst_x, dst_y):
        pltpu.sync_copy(masks.at[b, which, pl.ds(sid * _ROWS, _ROWS)], strip)

        def chunk(k, cnt):
            r = k // 14
            cb = (k % 14) * 16
            vals = strip[r, pl.ds(cb, 16)]
            m = vals != 0.0
            xs = (r + sid * _ROWS).astype(jnp.float32) + 0.0 * lane_f
            ys = cb.astype(jnp.float32) + lane_f
            mi = m.astype(jnp.int32)
            incl = plsc.cumsum(mi)
            idx = cnt + incl - mi
            plsc.store_scatter(dst_x, [idx], xs, mask=m)
            plsc.store_scatter(dst_y, [idx], ys, mask=m)
            return cnt + incl[15]

        cnt = lax.fori_loop(0, _CHUNKS, chunk, jnp.int32(0))
        sent = jnp.full((16,), _SENT, jnp.float32)
        all_on = jnp.full((16,), True)
        plsc.store_scatter(dst_x, [cnt + lane], sent, mask=all_on)
        plsc.store_scatter(dst_y, [cnt + lane], sent, mask=all_on)
        return cnt

    def upload(src_x, src_y, my_pad, c_off, dst_sx, dst_sy):
        def up(t, _):
            off = pl.multiple_of(c_off + t * 16, 16)
            src = pl.multiple_of(t * 16, 16)
            pltpu.sync_copy(src_x.at[pl.ds(src, 16)],
                            dst_sx.at[pl.ds(off, 16)])
            pltpu.sync_copy(src_y.at[pl.ds(src, 16)],
                            dst_sy.at[pl.ds(off, 16)])
            return 0
        lax.fori_loop(0, my_pad // 16, up, 0)

    def download(pad_n, src_sx, src_sy):
        def dn(t, _):
            off = pl.multiple_of(t * 512, 512)
            pltpu.sync_copy(src_sx.at[pl.ds(off, 512)],
                            cx.at[pl.ds(off, 512)])
            pltpu.sync_copy(src_sy.at[pl.ds(off, 512)],
                            cy.at[pl.ds(off, 512)])
            return 0
        lax.fori_loop(0, (pad_n + 511) // 512, dn, 0)

    def chamfer_dir(qx_ref, qy_ref, q_cnt, n_cand):
        nqc = (q_cnt + 15) // 16
        ncc = n_cand // 16

        def eval_chunk(jc, carry):
            b0, b1, b2, b3, ub, qs = carry
            (q0x, q0y), (q1x, q1y), (q2x, q2y), (q3x, q3y) = qs
            joff = pl.multiple_of(jc * 16, 16)
            cvx = cx[pl.ds(joff, 16)]
            cvy = cy[pl.ds(joff, 16)]
            for l in range(16):
                bx = jnp.broadcast_to(cvx[l], (16,))
                by = jnp.broadcast_to(cvy[l], (16,))
                d0x = q0x - bx
                d0y = q0y - by
                b0 = jnp.minimum(b0, d0x * d0x + d0y * d0y)
                d1x = q1x - bx
                d1y = q1y - by
                b1 = jnp.minimum(b1, d1x * d1x + d1y * d1y)
                d2x = q2x - bx
                d2y = q2y - by
                b2 = jnp.minimum(b2, d2x * d2x + d2y * d2y)
                d3x = q3x - bx
                d3y = q3y - by
                b3 = jnp.minimum(b3, d3x * d3x + d3y * d3y)
            ub = jnp.max(jnp.maximum(jnp.maximum(b0, b1),
                                     jnp.maximum(b2, b3)))
            return (b0, b1, b2, b3, ub, qs)

        def per_group(g, acc):
            qs = []
            for u in range(4):
                qi = jnp.minimum(g * 4 + u, nqc - 1)
                qoff = pl.multiple_of(qi * 16, 16)
                qs.append((qx_ref[pl.ds(qoff, 16)], qy_ref[pl.ds(qoff, 16)]))
            qs = tuple(qs)
            qmin = qs[0][0][0]
            mx = jnp.full((16,), 0.0)
            for u in range(4):
                mx = jnp.maximum(mx, jnp.where(qs[u][0] < _SENT * 0.5,
                                               qs[u][0], 0.0))
            qmax = jnp.max(mx)
            qmid = 0.5 * (qmin + qmax)

            def bs_cond(c):
                blo, bhi = c
                return bhi - blo > 1

            def bs_body(c):
                blo, bhi = c
                mid = (blo + bhi) // 2
                moff = pl.multiple_of(mid * 16, 16)
                xm = cx[pl.ds(moff, 16)][0]
                go = xm <= qmid
                return (jnp.where(go, mid, blo), jnp.where(go, bhi, mid))

            blo, _ = lax.while_loop(bs_cond, bs_body,
                                    (jnp.int32(0), ncc - 1))

            big = jnp.full((16,), _BIGD, jnp.float32)
            init = (big, big, big, big, jnp.float32(_BIGD), qs)

            def seed(carry):
                for dd in (-1, 0, 1):
                    jc = jnp.clip(blo + dd, 0, ncc - 1)
                    carry = eval_chunk(jc, carry)
                return carry

            carry = lax.cond(ncc > 0, seed, lambda c: c, init)

            def scan_chunk(jc, carry):
                joff = pl.multiple_of(jc * 16, 16)
                cvx = cx[pl.ds(joff, 16)]
                xlo = cvx[0]
                xhi = cvx[15]
                lb = jnp.maximum(jnp.maximum(xlo - qmax, qmin - xhi), 0.0)
                return lax.cond(lb * lb >= carry[4], lambda c: c,
                                lambda c: eval_chunk(jc, c), carry)

            b0, b1, b2, b3, _, _ = lax.fori_loop(0, ncc, scan_chunk, carry)

            dacc = jnp.float32(0.0)
            for u, b in enumerate((b0, b1, b2, b3)):
                s = _newton_sqrt(jnp.maximum(b, 1e-6))
                ok = (g * 4 + u < nqc) & (qs[u][0] < _SENT * 0.5)
                dacc = dacc + jnp.sum(jnp.where(ok, s, 0.0))
            return acc + dacc

        return lax.fori_loop(0, (nqc + 3) // 4, per_group, jnp.float32(0.0))

    def process_batch(bb, _):
        b = cid * 2 + bb
        cnt_p = compact(0, b, lx, ly)
        cnt_t = compact(1, b, tx, ty)

        cvec = jnp.where(lane == 0, cnt_p,
                         jnp.where(lane == 1, cnt_t, 0)).astype(jnp.int32)
        vbuf[...] = lax.bitcast_convert_type(cvec, jnp.float32)
        pltpu.sync_copy(vbuf, sh_cnts.at[sid])
        plsc.subcore_barrier()
        pltpu.sync_copy(sh_cnts, cnts_l)

        def scan_counts(i, carry):
            off_p, off_t, pad_p, pad_t = carry
            row = lax.bitcast_convert_type(cnts_l[i], jnp.int32)
            cp = jnp.sum(jnp.where(lane == 0, row, 0))
            ct = jnp.sum(jnp.where(lane == 1, row, 0))
            pp = ((cp + 15) // 16) * 16
            pt = ((ct + 15) // 16) * 16
            before = (i < sid).astype(jnp.int32)
            return (off_p + before * pp, off_t + before * pt,
                    pad_p + pp, pad_t + pt)

        z = jnp.int32(0)
        off_p, off_t, pad_p, pad_t = lax.fori_loop(
            0, _NS, scan_counts, (z, z, z, z))
        my_pad_p = ((cnt_p + 15) // 16) * 16
        my_pad_t = ((cnt_t + 15) // 16) * 16

        upload(lx, ly, my_pad_p, off_p, sh_x, sh_y)
        plsc.subcore_barrier()
        download(pad_p, sh_x, sh_y)
        accA = chamfer_dir(tx, ty, cnt_t, pad_p)
        plsc.subcore_barrier()

        upload(tx, ty, my_pad_t, off_t, sh_x, sh_y)
        plsc.subcore_barrier()
        download(pad_t, sh_x, sh_y)
        accB = chamfer_dir(lx, ly, cnt_p, pad_t)
        plsc.subcore_barrier()

        ovec = jnp.where(
            lane == 0, accA,
            jnp.where(lane == 1, accB,
                      jnp.where(lane == 2, cnt_p.astype(jnp.float32),
                                jnp.where(lane == 3,
                                          cnt_t.astype(jnp.float32), 0.0))))
        vbuf[...] = ovec
        pltpu.sync_copy(vbuf, out.at[b, sid])
        return 0

    lax.fori_loop(0, 2, process_batch, 0)


def _finish_body(parts_ref, out_ref):
    x = parts_ref[...]
    sumA = jnp.sum(x[:, :, 0], axis=1)
    sumB = jnp.sum(x[:, :, 1], axis=1)
    cp = jnp.sum(x[:, :, 2], axis=1)
    ct = jnp.sum(x[:, :, 3], axis=1)
    m1 = sumA / jnp.maximum(ct, 1.0)
    m2 = sumB / jnp.maximum(cp, 1.0)
    valid = (cp > 0.0) & (ct > 0.0)
    loss = jnp.sum(jnp.where(valid, 0.5 * (m1 + m2), 0.0))
    out_ref[...] = jnp.full((1, 1), loss)


def kernel(pred, target):
    if pred.ndim == 3:
        pred = pred[:, None]
    if target.ndim == 3:
        target = target[:, None]
    masks = pl.pallas_call(
        _masks_body,
        grid=(_B,),
        in_specs=[
            pl.BlockSpec((1, 1, _H, _W), lambda b: (b, 0, 0, 0)),
            pl.BlockSpec((1, 1, _H, _W), lambda b: (b, 0, 0, 0)),
        ],
        out_specs=pl.BlockSpec((1, 2, _HP, _W), lambda b: (b, 0, 0, 0)),
        out_shape=jax.ShapeDtypeStruct((_B, 2, _HP, _W), jnp.float32),
    )(pred.astype(jnp.float32), target.astype(jnp.float32))

    sc = pl.kernel(
        _sc_body,
        mesh=plsc.VectorSubcoreMesh(core_axis_name="c", subcore_axis_name="s"),
        compiler_params=pltpu.CompilerParams(needs_layout_passes=False),
        out_type=jax.ShapeDtypeStruct((_B, _NS, 16), jnp.float32),
        scratch_types=[
            pltpu.VMEM((_ROWS, _W), jnp.float32),
            pltpu.VMEM((_CAP,), jnp.float32),
            pltpu.VMEM((_CAP,), jnp.float32),
            pltpu.VMEM((_CAP,), jnp.float32),
            pltpu.VMEM((_CAP,), jnp.float32),
            pltpu.VMEM((_GCAP,), jnp.float32),
            pltpu.VMEM((_GCAP,), jnp.float32),
            pltpu.VMEM((_NS, 16), jnp.float32),
            pltpu.VMEM((16,), jnp.float32),
            pltpu.VMEM_SHARED((_NS, 16), jnp.float32),
            pltpu.VMEM_SHARED((_GCAP,), jnp.float32),
            pltpu.VMEM_SHARED((_GCAP,), jnp.float32),
        ],
    )(masks)

    out = pl.pallas_call(
        _finish_body,
        out_shape=jax.ShapeDtypeStruct((1, 1), jnp.float32),
    )(sc)
    return out.reshape(1)
